# Initial kernel scaffold; baseline (speedup 1.0000x reference)
import jax
import jax.numpy as jnp
from jax.experimental import pallas as pl


def kernel(x, edge_index, W1_rel, b1_rel, W1_root, W2_rel, b2_rel, W2_root):
    raise NotImplementedError("write your pallas kernel here")


# speedup vs baseline: 6.4634x; 6.4634x over previous
import functools

import jax
import jax.numpy as jnp
from jax import lax
from jax.experimental import pallas as pl
from jax.experimental.pallas import tpu as pltpu
from jax.experimental.pallas import tpu_sc as plsc

N_NODES = 50000
N_EDGES = 800000
HIDDEN = 64
SKEL = 256
FEAT = 16

NPAD = 50048
EPAD = 819200
EROWS = EPAD // 128
CHUNK_ROWS = 8
TILE_OUT = NPAD // 16


def _sc_segsum(table_ref, src_ref, dst_ref, zeros_ref, out_ref,
               srcv, dstv, rowsv, acc, gsem, ssem,
               *, n_phases, split_edges):
    c = lax.axis_index("c")
    s = lax.axis_index("s")
    zbase = s * TILE_OUT

    if split_edges:
        rows_per_tile = (EROWS // 2) // 16
        tile_row_base = c * (EROWS // 2) + s * rows_per_tile
    else:
        rows_per_tile = EROWS // 16
        tile_row_base = s * rows_per_tile
    chunks_per_tile = rows_per_tile // CHUNK_ROWS

    for p in range(n_phases):
        q = p * 2 + c
        table = table_ref.at[0 if table_ref.shape[0] == 1 else q]

        pltpu.sync_copy(zeros_ref.at[pl.ds(zbase, TILE_OUT)],
                        acc.at[pl.ds(zbase, TILE_OUT)])
        plsc.subcore_barrier()

        def chunk_body(i, carry):
            rbase = tile_row_base + i * CHUNK_ROWS
            pltpu.sync_copy(src_ref.at[pl.ds(rbase, CHUNK_ROWS)], srcv)
            pltpu.sync_copy(dst_ref.at[pl.ds(rbase, CHUNK_ROWS)], dstv)
            gd = [pltpu.async_copy(table.at[srcv.at[j]], rowsv.at[j], gsem)
                  for j in range(CHUNK_ROWS)]
            for d in gd:
                d.wait()
            sd = [pltpu.async_copy(rowsv.at[j], acc.at[dstv.at[j]], ssem,
                                   add=True)
                  for j in range(CHUNK_ROWS)]
            for d in sd:
                d.wait()
            return carry

        lax.fori_loop(0, chunks_per_tile, chunk_body, 0)
        plsc.subcore_barrier()

        pltpu.sync_copy(acc.at[pl.ds(zbase, TILE_OUT)],
                        out_ref.at[q].at[pl.ds(zbase, TILE_OUT)])


def _make_sc_segsum(table_planes, n_phases, split_edges):
    mesh = plsc.VectorSubcoreMesh(core_axis_name="c", subcore_axis_name="s")
    body = functools.partial(_sc_segsum, n_phases=n_phases,
                             split_edges=split_edges)
    return pl.kernel(
        body,
        out_type=jax.ShapeDtypeStruct((2 * n_phases, NPAD, FEAT), jnp.float32),
        mesh=mesh,
        scratch_types=[
            pltpu.VMEM((CHUNK_ROWS, 128), jnp.int32),
            pltpu.VMEM((CHUNK_ROWS, 128), jnp.int32),
            pltpu.VMEM((CHUNK_ROWS, 128, FEAT), jnp.float32),
            pltpu.VMEM_SHARED((NPAD, FEAT), jnp.float32),
            pltpu.SemaphoreType.DMA,
            pltpu.SemaphoreType.DMA,
        ],
        compiler_params=pltpu.CompilerParams(use_tc_tiling_on_sc=False),
    )


def _tc_layer1(aggp_ref, x_ref, wrel_ref, wroot_ref, b_ref, out_ref):
    agg = aggp_ref[0] + aggp_ref[1]
    h = (jnp.dot(agg, wrel_ref[...], preferred_element_type=jnp.float32)
         + jnp.dot(x_ref[...], wroot_ref[...], preferred_element_type=jnp.float32)
         + b_ref[...])
    h = jnp.maximum(h, 0.0)
    for q in range(4):
        out_ref[q] = h[:, q * FEAT:(q + 1) * FEAT]


def _tc_layer2(agg_ref, h_ref, wrel_ref, wroot_ref, b_ref, out_ref):
    agg = jnp.concatenate([agg_ref[q] for q in range(4)], axis=1)
    h = jnp.concatenate([h_ref[q] for q in range(4)], axis=1)
    out_ref[...] = (
        jnp.dot(agg, wrel_ref[...], preferred_element_type=jnp.float32)
        + jnp.dot(h, wroot_ref[...], preferred_element_type=jnp.float32)
        + b_ref[...])


_BLK = 2000
_GRID = N_NODES // _BLK


def kernel(x, edge_index, W1_rel, b1_rel, W1_root, W2_rel, b2_rel, W2_root):
    src = edge_index[0].astype(jnp.int32)
    dst = edge_index[1].astype(jnp.int32)
    npad_e = EPAD - N_EDGES
    srcp = jnp.concatenate([src, jnp.zeros((npad_e,), jnp.int32)])
    trash = N_NODES + (jnp.arange(npad_e, dtype=jnp.int32) % 8)
    dstp = jnp.concatenate([dst, trash])
    src2 = srcp.reshape(EROWS, 128)
    dst2 = dstp.reshape(EROWS, 128)

    x16 = jnp.pad(x, ((0, 0), (0, FEAT - 3)))
    w1rel = jnp.pad(W1_rel, ((0, FEAT - 3), (0, 0)))
    w1root = jnp.pad(W1_root, ((0, FEAT - 3), (0, 0)))
    zeros = jnp.zeros((NPAD, FEAT), jnp.float32)
    b1 = b1_rel.reshape(1, HIDDEN)
    b2 = b2_rel.reshape(1, SKEL)

    agg1p = _make_sc_segsum(1, n_phases=1, split_edges=True)(
        x16.reshape(1, N_NODES, FEAT), src2, dst2, zeros)

    hq = pl.pallas_call(
        _tc_layer1,
        grid=(_GRID,),
        in_specs=[
            pl.BlockSpec((2, _BLK, FEAT), lambda i: (0, i, 0)),
            pl.BlockSpec((_BLK, FEAT), lambda i: (i, 0)),
            pl.BlockSpec((FEAT, HIDDEN), lambda i: (0, 0)),
            pl.BlockSpec((FEAT, HIDDEN), lambda i: (0, 0)),
            pl.BlockSpec((1, HIDDEN), lambda i: (0, 0)),
        ],
        out_specs=pl.BlockSpec((4, _BLK, FEAT), lambda i: (0, i, 0)),
        out_shape=jax.ShapeDtypeStruct((4, N_NODES, FEAT), jnp.float32),
    )(agg1p, x16, w1rel, w1root, b1)

    agg2q = _make_sc_segsum(4, n_phases=2, split_edges=False)(
        hq, src2, dst2, zeros)

    out = pl.pallas_call(
        _tc_layer2,
        grid=(_GRID,),
        in_specs=[
            pl.BlockSpec((4, _BLK, FEAT), lambda i: (0, i, 0)),
            pl.BlockSpec((4, _BLK, FEAT), lambda i: (0, i, 0)),
            pl.BlockSpec((HIDDEN, SKEL), lambda i: (0, 0)),
            pl.BlockSpec((HIDDEN, SKEL), lambda i: (0, 0)),
            pl.BlockSpec((1, SKEL), lambda i: (0, 0)),
        ],
        out_specs=pl.BlockSpec((_BLK, SKEL), lambda i: (i, 0)),
        out_shape=jax.ShapeDtypeStruct((N_NODES, SKEL), jnp.float32),
    )(agg2q, hq, W2_rel, W2_root, b2)
    return out


# speedup vs baseline: 7.0896x; 1.0969x over previous
import functools

import jax
import jax.numpy as jnp
from jax import lax
from jax.experimental import pallas as pl
from jax.experimental.pallas import tpu as pltpu
from jax.experimental.pallas import tpu_sc as plsc

N_NODES = 50000
N_EDGES = 800000
HIDDEN = 64
SKEL = 256
FEAT = 16

NPAD = 50048
EPAD = 819200
EROWS = EPAD // 128
CHUNK_ROWS = 10
ECHUNKS = EROWS // CHUNK_ROWS
TILE_OUT = NPAD // 16


def _sc_segsum(table_ref, ech_ref, zeros_ref, out_ref,
               ebuf, rowsv, acc, lsem, gsem, ssem,
               *, n_phases, split_edges):
    c = lax.axis_index("c")
    s = lax.axis_index("s")
    zbase = s * TILE_OUT

    if split_edges:
        n = (ECHUNKS // 2) // 16
        chunk_base = c * (ECHUNKS // 2) + s * n
    else:
        n = ECHUNKS // 16
        chunk_base = s * n

    for p in range(n_phases):
        q = p * 2 + c
        table = table_ref.at[0 if table_ref.shape[0] == 1 else q]

        pltpu.sync_copy(zeros_ref.at[pl.ds(zbase, TILE_OUT)],
                        acc.at[pl.ds(zbase, TILE_OUT)])
        plsc.subcore_barrier()

        def idx_d(i):
            return pltpu.make_async_copy(
                ech_ref.at[chunk_base + i], ebuf.at[i % 2], lsem)

        def gather_ds(i):
            b = i % 2
            return [pltpu.make_async_copy(
                        table.at[ebuf.at[b, 0, j]], rowsv.at[b, j], gsem)
                    for j in range(CHUNK_ROWS)]

        def scatter_ds(i):
            b = i % 2
            return [pltpu.make_async_copy(
                        rowsv.at[b, j], acc.at[ebuf.at[b, 1, j]], ssem)
                    for j in range(CHUNK_ROWS)]

        def fire_scatters(i):
            b = i % 2
            for j in range(CHUNK_ROWS):
                pltpu.async_copy(rowsv.at[b, j], acc.at[ebuf.at[b, 1, j]],
                                 ssem, add=True)

        def sub(i, carry):
            for d in gather_ds(i):
                d.wait()
            for d in scatter_ds(i - 1):
                d.wait()
            idx_d(i + 1).start()
            fire_scatters(i)
            idx_d(i + 1).wait()
            for d in gather_ds(i + 1):
                d.start()
            return carry

        idx_d(0).start()
        idx_d(0).wait()
        for d in gather_ds(0):
            d.start()
        idx_d(1).start()
        for d in gather_ds(0):
            d.wait()
        fire_scatters(0)
        idx_d(1).wait()
        for d in gather_ds(1):
            d.start()
        lax.fori_loop(1, n - 1, sub, 0)
        for d in gather_ds(n - 1):
            d.wait()
        for d in scatter_ds(n - 2):
            d.wait()
        fire_scatters(n - 1)
        for d in scatter_ds(n - 1):
            d.wait()

        plsc.subcore_barrier()
        pltpu.sync_copy(acc.at[pl.ds(zbase, TILE_OUT)],
                        out_ref.at[q].at[pl.ds(zbase, TILE_OUT)])


def _make_sc_segsum(table_planes, n_phases, split_edges):
    mesh = plsc.VectorSubcoreMesh(core_axis_name="c", subcore_axis_name="s")
    body = functools.partial(_sc_segsum, n_phases=n_phases,
                             split_edges=split_edges)
    return pl.kernel(
        body,
        out_type=jax.ShapeDtypeStruct((2 * n_phases, NPAD, FEAT), jnp.float32),
        mesh=mesh,
        scratch_types=[
            pltpu.VMEM((2, 2, CHUNK_ROWS, 128), jnp.int32),
            pltpu.VMEM((2, CHUNK_ROWS, 128, FEAT), jnp.float32),
            pltpu.VMEM_SHARED((NPAD, FEAT), jnp.float32),
            pltpu.SemaphoreType.DMA,
            pltpu.SemaphoreType.DMA,
            pltpu.SemaphoreType.DMA,
        ],
        compiler_params=pltpu.CompilerParams(use_tc_tiling_on_sc=False),
    )


def _tc_layer1(aggp_ref, x_ref, wrel_ref, wroot_ref, b_ref, out_ref):
    agg = aggp_ref[0] + aggp_ref[1]
    h = (jnp.dot(agg, wrel_ref[...], preferred_element_type=jnp.float32)
         + jnp.dot(x_ref[...], wroot_ref[...], preferred_element_type=jnp.float32)
         + b_ref[...])
    h = jnp.maximum(h, 0.0)
    for q in range(4):
        out_ref[q] = h[:, q * FEAT:(q + 1) * FEAT]


def _tc_layer2(agg_ref, h_ref, wrel_ref, wroot_ref, b_ref, out_ref):
    agg = jnp.concatenate([agg_ref[q] for q in range(4)], axis=1)
    h = jnp.concatenate([h_ref[q] for q in range(4)], axis=1)
    out_ref[...] = (
        jnp.dot(agg, wrel_ref[...], preferred_element_type=jnp.float32)
        + jnp.dot(h, wroot_ref[...], preferred_element_type=jnp.float32)
        + b_ref[...])


_BLK = 2000
_GRID = N_NODES // _BLK


def kernel(x, edge_index, W1_rel, b1_rel, W1_root, W2_rel, b2_rel, W2_root):
    src = edge_index[0].astype(jnp.int32)
    dst = edge_index[1].astype(jnp.int32)
    npad_e = EPAD - N_EDGES
    srcp = jnp.concatenate([src, jnp.zeros((npad_e,), jnp.int32)])
    trash = N_NODES + (jnp.arange(npad_e, dtype=jnp.int32) % 8)
    dstp = jnp.concatenate([dst, trash])
    ech = jnp.stack([srcp.reshape(ECHUNKS, CHUNK_ROWS, 128),
                     dstp.reshape(ECHUNKS, CHUNK_ROWS, 128)], axis=1)

    x16 = jnp.pad(x, ((0, 0), (0, FEAT - 3)))
    w1rel = jnp.pad(W1_rel, ((0, FEAT - 3), (0, 0)))
    w1root = jnp.pad(W1_root, ((0, FEAT - 3), (0, 0)))
    zeros = jnp.zeros((NPAD, FEAT), jnp.float32)
    b1 = b1_rel.reshape(1, HIDDEN)
    b2 = b2_rel.reshape(1, SKEL)

    agg1p = _make_sc_segsum(1, n_phases=1, split_edges=True)(
        x16.reshape(1, N_NODES, FEAT), ech, zeros)

    hq = pl.pallas_call(
        _tc_layer1,
        grid=(_GRID,),
        in_specs=[
            pl.BlockSpec((2, _BLK, FEAT), lambda i: (0, i, 0)),
            pl.BlockSpec((_BLK, FEAT), lambda i: (i, 0)),
            pl.BlockSpec((FEAT, HIDDEN), lambda i: (0, 0)),
            pl.BlockSpec((FEAT, HIDDEN), lambda i: (0, 0)),
            pl.BlockSpec((1, HIDDEN), lambda i: (0, 0)),
        ],
        out_specs=pl.BlockSpec((4, _BLK, FEAT), lambda i: (0, i, 0)),
        out_shape=jax.ShapeDtypeStruct((4, N_NODES, FEAT), jnp.float32),
    )(agg1p, x16, w1rel, w1root, b1)

    agg2q = _make_sc_segsum(4, n_phases=2, split_edges=False)(
        hq, ech, zeros)

    out = pl.pallas_call(
        _tc_layer2,
        grid=(_GRID,),
        in_specs=[
            pl.BlockSpec((4, _BLK, FEAT), lambda i: (0, i, 0)),
            pl.BlockSpec((4, _BLK, FEAT), lambda i: (0, i, 0)),
            pl.BlockSpec((HIDDEN, SKEL), lambda i: (0, 0)),
            pl.BlockSpec((HIDDEN, SKEL), lambda i: (0, 0)),
            pl.BlockSpec((1, SKEL), lambda i: (0, 0)),
        ],
        out_specs=pl.BlockSpec((_BLK, SKEL), lambda i: (i, 0)),
        out_shape=jax.ShapeDtypeStruct((N_NODES, SKEL), jnp.float32),
    )(agg2q, hq, W2_rel, W2_root, b2)
    return out


# speedup vs baseline: 7.1458x; 1.0079x over previous
import functools

import jax
import jax.numpy as jnp
from jax import lax
from jax.experimental import pallas as pl
from jax.experimental.pallas import tpu as pltpu
from jax.experimental.pallas import tpu_sc as plsc

N_NODES = 50000
N_EDGES = 800000
HIDDEN = 64
SKEL = 256
FEAT = 16

NPAD = 50048
EPAD = 819200
SLEN = 512
NSTR = 2
ECHUNKS = EPAD // (SLEN * NSTR)
TILE_OUT = NPAD // 16


def _sc_segsum(table_ref, ech_ref, zeros_ref, out_ref,
               ebuf, rowsv, acc, lsem, gsem, ssem,
               *, n_phases, split_edges):
    c = lax.axis_index("c")
    s = lax.axis_index("s")
    zbase = s * TILE_OUT

    if split_edges:
        n = (ECHUNKS // 2) // 16
        chunk_base = c * (ECHUNKS // 2) + s * n
    else:
        n = ECHUNKS // 16
        chunk_base = s * n

    for p in range(n_phases):
        q = p * 2 + c
        table = table_ref.at[0 if table_ref.shape[0] == 1 else q]

        pltpu.sync_copy(zeros_ref.at[pl.ds(zbase, TILE_OUT)],
                        acc.at[pl.ds(zbase, TILE_OUT)])
        plsc.subcore_barrier()

        def idx_d(i):
            return pltpu.make_async_copy(
                ech_ref.at[chunk_base + i], ebuf.at[i % 2], lsem)

        def gather_ds(i):
            b = i % 2
            return [pltpu.make_async_copy(
                        table.at[ebuf.at[b, 0, j]], rowsv.at[b, j], gsem)
                    for j in range(NSTR)]

        def scatter_ds(i):
            b = i % 2
            return [pltpu.make_async_copy(
                        rowsv.at[b, j], acc.at[ebuf.at[b, 1, j]], ssem)
                    for j in range(NSTR)]

        def fire_scatters(i):
            b = i % 2
            for j in range(NSTR):
                pltpu.async_copy(rowsv.at[b, j], acc.at[ebuf.at[b, 1, j]],
                                 ssem, add=True)

        def sub(i, carry):
            for d in gather_ds(i):
                d.wait()
            for d in scatter_ds(i - 1):
                d.wait()
            idx_d(i + 1).start()
            fire_scatters(i)
            idx_d(i + 1).wait()
            for d in gather_ds(i + 1):
                d.start()
            return carry

        idx_d(0).start()
        idx_d(0).wait()
        for d in gather_ds(0):
            d.start()
        idx_d(1).start()
        for d in gather_ds(0):
            d.wait()
        fire_scatters(0)
        idx_d(1).wait()
        for d in gather_ds(1):
            d.start()
        lax.fori_loop(1, n - 1, sub, 0)
        for d in gather_ds(n - 1):
            d.wait()
        for d in scatter_ds(n - 2):
            d.wait()
        fire_scatters(n - 1)
        for d in scatter_ds(n - 1):
            d.wait()

        plsc.subcore_barrier()
        pltpu.sync_copy(acc.at[pl.ds(zbase, TILE_OUT)],
                        out_ref.at[q].at[pl.ds(zbase, TILE_OUT)])


def _make_sc_segsum(table_planes, n_phases, split_edges):
    mesh = plsc.VectorSubcoreMesh(core_axis_name="c", subcore_axis_name="s")
    body = functools.partial(_sc_segsum, n_phases=n_phases,
                             split_edges=split_edges)
    return pl.kernel(
        body,
        out_type=jax.ShapeDtypeStruct((2 * n_phases, NPAD, FEAT), jnp.float32),
        mesh=mesh,
        scratch_types=[
            pltpu.VMEM((2, 2, NSTR, SLEN), jnp.int32),
            pltpu.VMEM((2, NSTR, SLEN, FEAT), jnp.float32),
            pltpu.VMEM_SHARED((NPAD, FEAT), jnp.float32),
            pltpu.SemaphoreType.DMA,
            pltpu.SemaphoreType.DMA,
            pltpu.SemaphoreType.DMA,
        ],
        compiler_params=pltpu.CompilerParams(use_tc_tiling_on_sc=False),
    )


def _tc_layer1(aggp_ref, x_ref, wrel_ref, wroot_ref, b_ref, out_ref):
    agg = aggp_ref[0] + aggp_ref[1]
    h = (jnp.dot(agg, wrel_ref[...], preferred_element_type=jnp.float32)
         + jnp.dot(x_ref[...], wroot_ref[...], preferred_element_type=jnp.float32)
         + b_ref[...])
    h = jnp.maximum(h, 0.0)
    for q in range(4):
        out_ref[q] = h[:, q * FEAT:(q + 1) * FEAT]


def _tc_layer2(agg_ref, h_ref, wrel_ref, wroot_ref, b_ref, out_ref):
    agg = jnp.concatenate([agg_ref[q] for q in range(4)], axis=1)
    h = jnp.concatenate([h_ref[q] for q in range(4)], axis=1)
    out_ref[...] = (
        jnp.dot(agg, wrel_ref[...], preferred_element_type=jnp.float32)
        + jnp.dot(h, wroot_ref[...], preferred_element_type=jnp.float32)
        + b_ref[...])


_BLK = 2000
_GRID = N_NODES // _BLK


def kernel(x, edge_index, W1_rel, b1_rel, W1_root, W2_rel, b2_rel, W2_root):
    src = edge_index[0].astype(jnp.int32)
    dst = edge_index[1].astype(jnp.int32)
    npad_e = EPAD - N_EDGES
    srcp = jnp.concatenate([src, jnp.zeros((npad_e,), jnp.int32)])
    trash = N_NODES + (jnp.arange(npad_e, dtype=jnp.int32) % 8)
    dstp = jnp.concatenate([dst, trash])
    ech = jnp.stack([srcp.reshape(ECHUNKS, NSTR, SLEN),
                     dstp.reshape(ECHUNKS, NSTR, SLEN)], axis=1)

    x16 = jnp.pad(x, ((0, 0), (0, FEAT - 3)))
    w1rel = jnp.pad(W1_rel, ((0, FEAT - 3), (0, 0)))
    w1root = jnp.pad(W1_root, ((0, FEAT - 3), (0, 0)))
    zeros = jnp.zeros((NPAD, FEAT), jnp.float32)
    b1 = b1_rel.reshape(1, HIDDEN)
    b2 = b2_rel.reshape(1, SKEL)

    agg1p = _make_sc_segsum(1, n_phases=1, split_edges=True)(
        x16.reshape(1, N_NODES, FEAT), ech, zeros)

    hq = pl.pallas_call(
        _tc_layer1,
        grid=(_GRID,),
        in_specs=[
            pl.BlockSpec((2, _BLK, FEAT), lambda i: (0, i, 0)),
            pl.BlockSpec((_BLK, FEAT), lambda i: (i, 0)),
            pl.BlockSpec((FEAT, HIDDEN), lambda i: (0, 0)),
            pl.BlockSpec((FEAT, HIDDEN), lambda i: (0, 0)),
            pl.BlockSpec((1, HIDDEN), lambda i: (0, 0)),
        ],
        out_specs=pl.BlockSpec((4, _BLK, FEAT), lambda i: (0, i, 0)),
        out_shape=jax.ShapeDtypeStruct((4, N_NODES, FEAT), jnp.float32),
    )(agg1p, x16, w1rel, w1root, b1)

    agg2q = _make_sc_segsum(4, n_phases=2, split_edges=False)(
        hq, ech, zeros)

    out = pl.pallas_call(
        _tc_layer2,
        grid=(_GRID,),
        in_specs=[
            pl.BlockSpec((4, _BLK, FEAT), lambda i: (0, i, 0)),
            pl.BlockSpec((4, _BLK, FEAT), lambda i: (0, i, 0)),
            pl.BlockSpec((HIDDEN, SKEL), lambda i: (0, 0)),
            pl.BlockSpec((HIDDEN, SKEL), lambda i: (0, 0)),
            pl.BlockSpec((1, SKEL), lambda i: (0, 0)),
        ],
        out_specs=pl.BlockSpec((_BLK, SKEL), lambda i: (i, 0)),
        out_shape=jax.ShapeDtypeStruct((N_NODES, SKEL), jnp.float32),
    )(agg2q, hq, W2_rel, W2_root, b2)
    return out


# speedup vs baseline: 7.1604x; 1.0020x over previous
import functools

import jax
import jax.numpy as jnp
from jax import lax
from jax.experimental import pallas as pl
from jax.experimental.pallas import tpu as pltpu
from jax.experimental.pallas import tpu_sc as plsc

N_NODES = 50000
HIDDEN = 64
SKEL = 256
FEAT = 16

NPAD = 51200
PROWS = NPAD * FEAT // 128
N_EDGES = 800000
EPAD = 819200
SLEN = 512
NSTR = 2
ECHUNKS = EPAD // (SLEN * NSTR)
TILE_OUT = NPAD // 16


def _sc_segsum(table_ref, ech_ref, zeros_ref, out_ref,
               ebuf, gidx, rowsv, acc, lsem, gsem, ssem,
               *, n_phases, split_edges, scale4):
    c = lax.axis_index("c")
    s = lax.axis_index("s")
    zbase = s * TILE_OUT
    table = table_ref
    ech = ech_ref
    zv = zeros_ref
    outv = out_ref

    if split_edges:
        n = (ECHUNKS // 2) // 16
        chunk_base = c * (ECHUNKS // 2) + s * n
    else:
        n = ECHUNKS // 16
        chunk_base = s * n

    for p in range(n_phases):
        q = p * 2 + c

        pltpu.sync_copy(zv.at[pl.ds(zbase, TILE_OUT)],
                        acc.at[pl.ds(zbase, TILE_OUT)])
        plsc.subcore_barrier()

        def idx_d(i):
            return pltpu.make_async_copy(
                ech.at[chunk_base + i], ebuf.at[i % 2], lsem)

        def make_gidx(i):
            b = i % 2
            for j in range(NSTR):
                for k in range(SLEN // 16):
                    v = ebuf[b, j, pl.ds(16 * k, 16)]
                    gidx[b, j, pl.ds(16 * k, 16)] = v * 4 + q

        def gather_ds(i):
            b = i % 2
            iref = gidx if scale4 else ebuf
            return [pltpu.make_async_copy(
                        table.at[iref.at[b, j]], rowsv.at[b, j], gsem)
                    for j in range(NSTR)]

        def scatter_ds(i):
            b = i % 2
            return [pltpu.make_async_copy(
                        rowsv.at[b, j], acc.at[ebuf.at[b, NSTR + j]], ssem)
                    for j in range(NSTR)]

        def fire_scatters(i):
            b = i % 2
            for j in range(NSTR):
                pltpu.async_copy(rowsv.at[b, j], acc.at[ebuf.at[b, NSTR + j]],
                                 ssem, add=True)

        def fire_gathers(i):
            if scale4:
                make_gidx(i)
            for d in gather_ds(i):
                d.start()

        def sub(i, carry):
            for d in gather_ds(i):
                d.wait()
            for d in scatter_ds(i - 1):
                d.wait()
            idx_d(i + 1).start()
            fire_scatters(i)
            idx_d(i + 1).wait()
            fire_gathers(i + 1)
            return carry

        idx_d(0).start()
        idx_d(0).wait()
        fire_gathers(0)
        idx_d(1).start()
        for d in gather_ds(0):
            d.wait()
        fire_scatters(0)
        idx_d(1).wait()
        fire_gathers(1)
        lax.fori_loop(1, n - 1, sub, 0)
        for d in gather_ds(n - 1):
            d.wait()
        for d in scatter_ds(n - 2):
            d.wait()
        fire_scatters(n - 1)
        for d in scatter_ds(n - 1):
            d.wait()

        plsc.subcore_barrier()
        pltpu.sync_copy(acc.at[pl.ds(zbase, TILE_OUT)],
                        outv.at[q].at[pl.ds(zbase, TILE_OUT)])


def _make_sc_segsum(n_phases, split_edges, scale4):
    mesh = plsc.VectorSubcoreMesh(core_axis_name="c", subcore_axis_name="s")
    body = functools.partial(_sc_segsum, n_phases=n_phases,
                             split_edges=split_edges, scale4=scale4)
    return pl.kernel(
        body,
        out_type=jax.ShapeDtypeStruct((2 * n_phases, NPAD, FEAT),
                                      jnp.float32),
        mesh=mesh,
        scratch_types=[
            pltpu.VMEM((2, 2 * NSTR, SLEN), jnp.int32),
            pltpu.VMEM((2, NSTR, SLEN), jnp.int32),
            pltpu.VMEM((2, NSTR, SLEN, FEAT), jnp.float32),
            pltpu.VMEM_SHARED((NPAD, FEAT), jnp.float32),
            pltpu.SemaphoreType.DMA,
            pltpu.SemaphoreType.DMA,
            pltpu.SemaphoreType.DMA,
        ],
        compiler_params=pltpu.CompilerParams(use_tc_tiling_on_sc=False),
    )


def _tc_layer1(aggp_ref, x_ref, wrel_ref, wroot_ref, b_ref, out_ref):
    a = aggp_ref[0] + aggp_ref[1]
    x = x_ref[...]
    for m in range(8):
        am = a[:, m * FEAT:(m + 1) * FEAT]
        xm = x[:, m * FEAT:(m + 1) * FEAT]
        h = (jnp.dot(am, wrel_ref[...], preferred_element_type=jnp.float32)
             + jnp.dot(xm, wroot_ref[...], preferred_element_type=jnp.float32)
             + b_ref[...])
        out_ref[:, m * HIDDEN:(m + 1) * HIDDEN] = jnp.maximum(h, 0.0)


def _tc_layer2(agg_ref, h_ref, wrel_ref, wroot_ref, b_ref, out_ref):
    h = h_ref[...]
    for m in range(8):
        am = jnp.concatenate(
            [agg_ref[qq][:, m * FEAT:(m + 1) * FEAT] for qq in range(4)],
            axis=1)
        hm = h[:, m * HIDDEN:(m + 1) * HIDDEN]
        out_ref[:, m * SKEL:(m + 1) * SKEL] = (
            jnp.dot(am, wrel_ref[...], preferred_element_type=jnp.float32)
            + jnp.dot(hm, wroot_ref[...], preferred_element_type=jnp.float32)
            + b_ref[...])


_PBLK = 400
_GRID = PROWS // _PBLK


def kernel(x, edge_index, W1_rel, b1_rel, W1_root, W2_rel, b2_rel, W2_root):
    src = edge_index[0].astype(jnp.int32)
    dst = edge_index[1].astype(jnp.int32)
    npad_e = EPAD - N_EDGES
    srcp = jnp.concatenate([src, jnp.zeros((npad_e,), jnp.int32)])
    trash = N_NODES + (jnp.arange(npad_e, dtype=jnp.int32) % 1024)
    dstp = jnp.concatenate([dst, trash])
    ech = jnp.stack([srcp.reshape(ECHUNKS, NSTR, SLEN),
                     dstp.reshape(ECHUNKS, NSTR, SLEN)],
                    axis=1).reshape(ECHUNKS, 2 * NSTR, SLEN)

    xv = jnp.pad(x, ((0, NPAD - N_NODES), (0, FEAT - 3)))
    xp = xv.reshape(PROWS, 128)
    w1rel = jnp.pad(W1_rel, ((0, FEAT - 3), (0, 0)))
    w1root = jnp.pad(W1_root, ((0, FEAT - 3), (0, 0)))
    zeros = jnp.zeros((NPAD, FEAT), jnp.float32)
    b1 = b1_rel.reshape(1, HIDDEN)
    b2 = b2_rel.reshape(1, SKEL)

    agg1p = _make_sc_segsum(n_phases=1, split_edges=True, scale4=False)(
        xv, ech, zeros)

    hp = pl.pallas_call(
        _tc_layer1,
        grid=(_GRID,),
        in_specs=[
            pl.BlockSpec((2, _PBLK, 128), lambda i: (0, i, 0)),
            pl.BlockSpec((_PBLK, 128), lambda i: (i, 0)),
            pl.BlockSpec((FEAT, HIDDEN), lambda i: (0, 0)),
            pl.BlockSpec((FEAT, HIDDEN), lambda i: (0, 0)),
            pl.BlockSpec((1, HIDDEN), lambda i: (0, 0)),
        ],
        out_specs=pl.BlockSpec((_PBLK, 512), lambda i: (i, 0)),
        out_shape=jax.ShapeDtypeStruct((PROWS, 512), jnp.float32),
    )(agg1p.reshape(2, PROWS, 128), xp, w1rel, w1root, b1)

    agg2q = _make_sc_segsum(n_phases=2, split_edges=False, scale4=True)(
        hp.reshape(4 * NPAD, FEAT), ech, zeros)

    outp = pl.pallas_call(
        _tc_layer2,
        grid=(_GRID,),
        in_specs=[
            pl.BlockSpec((4, _PBLK, 128), lambda i: (0, i, 0)),
            pl.BlockSpec((_PBLK, 512), lambda i: (i, 0)),
            pl.BlockSpec((HIDDEN, SKEL), lambda i: (0, 0)),
            pl.BlockSpec((HIDDEN, SKEL), lambda i: (0, 0)),
            pl.BlockSpec((1, SKEL), lambda i: (0, 0)),
        ],
        out_specs=pl.BlockSpec((_PBLK, 8 * SKEL), lambda i: (i, 0)),
        out_shape=jax.ShapeDtypeStruct((PROWS, 8 * SKEL), jnp.float32),
    )(agg2q.reshape(4, PROWS, 128), hp, W2_rel, W2_root, b2)
    return outp.reshape(NPAD, SKEL)[:N_NODES]


# speedup vs baseline: 8.4241x; 1.1765x over previous
import functools

import jax
import jax.numpy as jnp
from jax import lax
from jax.experimental import pallas as pl
from jax.experimental.pallas import tpu as pltpu
from jax.experimental.pallas import tpu_sc as plsc

N_NODES = 50000
HIDDEN = 64
SKEL = 256
FEAT = 16

NPAD = 51200
PROWS = NPAD * FEAT // 128
N_EDGES = 800000
EPAD = 819200
SLEN = 512
NSTR = 2
ECHUNKS = EPAD // (SLEN * NSTR)
TILE_OUT = NPAD // 16


def _sc_segsum(table_ref, src_ref, dst_ref, zeros_ref, out_ref,
               ebs, ebd, rowsv, acc, lsem, gsem, ssem,
               *, n_phases, split_edges):
    c = lax.axis_index("c")
    s = lax.axis_index("s")
    zbase = s * TILE_OUT

    if split_edges:
        n = (ECHUNKS // 2) // 16
        chunk_base = c * (ECHUNKS // 2) + s * n
    else:
        n = ECHUNKS // 16
        chunk_base = s * n

    for p in range(n_phases):
        q = p * 2 + c
        table = table_ref if table_ref.ndim == 2 else table_ref.at[q]

        pltpu.sync_copy(zeros_ref.at[pl.ds(zbase, TILE_OUT)],
                        acc.at[pl.ds(zbase, TILE_OUT)])
        plsc.subcore_barrier()

        def idx_ds(i):
            rb = (chunk_base + i) * NSTR
            return [pltpu.make_async_copy(
                        src_ref.at[pl.ds(rb, NSTR)], ebs.at[i % 2], lsem),
                    pltpu.make_async_copy(
                        dst_ref.at[pl.ds(rb, NSTR)], ebd.at[i % 2], lsem)]

        def gather_ds(i):
            b = i % 2
            return [pltpu.make_async_copy(
                        table.at[ebs.at[b, j]], rowsv.at[b, j], gsem)
                    for j in range(NSTR)]

        def scatter_ds(i):
            b = i % 2
            return [pltpu.make_async_copy(
                        rowsv.at[b, j], acc.at[ebd.at[b, j]], ssem)
                    for j in range(NSTR)]

        def fire_scatters(i):
            b = i % 2
            for j in range(NSTR):
                pltpu.async_copy(rowsv.at[b, j], acc.at[ebd.at[b, j]],
                                 ssem, add=True)

        def sub(i, carry):
            for d in gather_ds(i):
                d.wait()
            for d in scatter_ds(i - 1):
                d.wait()
            for d in idx_ds(i + 1):
                d.start()
            fire_scatters(i)
            for d in idx_ds(i + 1):
                d.wait()
            for d in gather_ds(i + 1):
                d.start()
            return carry

        for d in idx_ds(0):
            d.start()
        for d in idx_ds(0):
            d.wait()
        for d in gather_ds(0):
            d.start()
        for d in idx_ds(1):
            d.start()
        for d in gather_ds(0):
            d.wait()
        fire_scatters(0)
        for d in idx_ds(1):
            d.wait()
        for d in gather_ds(1):
            d.start()
        lax.fori_loop(1, n - 1, sub, 0)
        for d in gather_ds(n - 1):
            d.wait()
        for d in scatter_ds(n - 2):
            d.wait()
        fire_scatters(n - 1)
        for d in scatter_ds(n - 1):
            d.wait()

        plsc.subcore_barrier()
        pltpu.sync_copy(acc.at[pl.ds(zbase, TILE_OUT)],
                        out_ref.at[q].at[pl.ds(zbase, TILE_OUT)])


def _make_sc_segsum(n_phases, split_edges):
    mesh = plsc.VectorSubcoreMesh(core_axis_name="c", subcore_axis_name="s")
    body = functools.partial(_sc_segsum, n_phases=n_phases,
                             split_edges=split_edges)
    return pl.kernel(
        body,
        out_type=jax.ShapeDtypeStruct((2 * n_phases, NPAD, FEAT),
                                      jnp.float32),
        mesh=mesh,
        scratch_types=[
            pltpu.VMEM((2, NSTR, SLEN), jnp.int32),
            pltpu.VMEM((2, NSTR, SLEN), jnp.int32),
            pltpu.VMEM((2, NSTR, SLEN, FEAT), jnp.float32),
            pltpu.VMEM_SHARED((NPAD, FEAT), jnp.float32),
            pltpu.SemaphoreType.DMA,
            pltpu.SemaphoreType.DMA,
            pltpu.SemaphoreType.DMA,
        ],
        compiler_params=pltpu.CompilerParams(use_tc_tiling_on_sc=False),
    )


def _tc_layer1(aggp_ref, x_ref, wrel_ref, wroot_ref, b_ref, out_ref):
    a = aggp_ref[0] + aggp_ref[1]
    x = x_ref[...]
    for m in range(8):
        am = a[:, m * FEAT:(m + 1) * FEAT]
        xm = x[:, m * FEAT:(m + 1) * FEAT]
        h = (jnp.dot(am, wrel_ref[...], preferred_element_type=jnp.float32)
             + jnp.dot(xm, wroot_ref[...], preferred_element_type=jnp.float32)
             + b_ref[...])
        h = jnp.maximum(h, 0.0)
        for qq in range(4):
            out_ref[qq, :, m * FEAT:(m + 1) * FEAT] = (
                h[:, qq * FEAT:(qq + 1) * FEAT])


def _tc_layer2(agg_ref, h_ref, wrel_ref, wroot_ref, b_ref, out_ref):
    rs = []
    for m in range(8):
        am = jnp.concatenate(
            [agg_ref[qq][:, m * FEAT:(m + 1) * FEAT] for qq in range(4)],
            axis=1)
        hm = jnp.concatenate(
            [h_ref[qq][:, m * FEAT:(m + 1) * FEAT] for qq in range(4)],
            axis=1)
        rs.append(
            jnp.dot(am, wrel_ref[...], preferred_element_type=jnp.float32)
            + jnp.dot(hm, wroot_ref[...], preferred_element_type=jnp.float32)
            + b_ref[...])
    blk = rs[0].shape[0]
    out_ref[...] = jnp.stack(rs, axis=1).reshape(blk * 8, SKEL)


_PBLK = 400
_GRID = PROWS // _PBLK


def kernel(x, edge_index, W1_rel, b1_rel, W1_root, W2_rel, b2_rel, W2_root):
    src = edge_index[0].astype(jnp.int32)
    dst = edge_index[1].astype(jnp.int32)
    npad_e = EPAD - N_EDGES
    srcp = jnp.concatenate([src, jnp.zeros((npad_e,), jnp.int32)])
    trash = N_NODES + (jnp.arange(npad_e, dtype=jnp.int32) % 1024)
    dstp = jnp.concatenate([dst, trash])
    src2 = srcp.reshape(ECHUNKS * NSTR, SLEN)
    dst2 = dstp.reshape(ECHUNKS * NSTR, SLEN)

    xv = jnp.pad(x, ((0, NPAD - N_NODES), (0, FEAT - 3)))
    xp = xv.reshape(PROWS, 128)
    w1rel = jnp.pad(W1_rel, ((0, FEAT - 3), (0, 0)))
    w1root = jnp.pad(W1_root, ((0, FEAT - 3), (0, 0)))
    zeros = jnp.zeros((NPAD, FEAT), jnp.float32)
    b1 = b1_rel.reshape(1, HIDDEN)
    b2 = b2_rel.reshape(1, SKEL)

    agg1p = _make_sc_segsum(n_phases=1, split_edges=True)(
        xv, src2, dst2, zeros)

    hq4 = pl.pallas_call(
        _tc_layer1,
        grid=(_GRID,),
        in_specs=[
            pl.BlockSpec((2, _PBLK, 128), lambda i: (0, i, 0)),
            pl.BlockSpec((_PBLK, 128), lambda i: (i, 0)),
            pl.BlockSpec((FEAT, HIDDEN), lambda i: (0, 0)),
            pl.BlockSpec((FEAT, HIDDEN), lambda i: (0, 0)),
            pl.BlockSpec((1, HIDDEN), lambda i: (0, 0)),
        ],
        out_specs=pl.BlockSpec((4, _PBLK, 128), lambda i: (0, i, 0)),
        out_shape=jax.ShapeDtypeStruct((4, PROWS, 128), jnp.float32),
    )(agg1p.reshape(2, PROWS, 128), xp, w1rel, w1root, b1)

    agg2q = _make_sc_segsum(n_phases=2, split_edges=False)(
        hq4.reshape(4, NPAD, FEAT), src2, dst2, zeros)

    out = pl.pallas_call(
        _tc_layer2,
        grid=(_GRID,),
        in_specs=[
            pl.BlockSpec((4, _PBLK, 128), lambda i: (0, i, 0)),
            pl.BlockSpec((4, _PBLK, 128), lambda i: (0, i, 0)),
            pl.BlockSpec((HIDDEN, SKEL), lambda i: (0, 0)),
            pl.BlockSpec((HIDDEN, SKEL), lambda i: (0, 0)),
            pl.BlockSpec((1, SKEL), lambda i: (0, 0)),
        ],
        out_specs=pl.BlockSpec((_PBLK * 8, SKEL), lambda i: (i, 0)),
        out_shape=jax.ShapeDtypeStruct((N_NODES, SKEL), jnp.float32),
    )(agg2q.reshape(4, PROWS, 128), hq4, W2_rel, W2_root, b2)
    return out
